# baseline (speedup 1.0000x reference)
import dataclasses
import functools

import jax
import jax.numpy as jnp
from jax import lax
from jax.experimental import pallas as pl
from jax.experimental.pallas import tpu as pltpu
from jax.experimental.pallas import tpu_sc as plsc

_N = 10000
_NPAD = 10240
_PADROW = 10016
_E = 320000
_D = 128
_T = 4

_C = 80
_NSUB = 16
_ROWS_PER_TILE = _NPAD // _NSUB
_PER_SUB = _E // _NSUB // _C
_REC = 4
_CW = _C * _REC
_SEGC = 50
_NSEG = _PER_SUB // _SEGC
_LCAP = _SEGC * _C + 2 * _C

_TRASH = _PADROW << 16


def _sc_agg(x, edata, nt_flat):
    mesh = plsc.VectorSubcoreMesh(core_axis_name="c", subcore_axis_name="s")
    cp = pltpu.CompilerParams(use_tc_tiling_on_sc=False)
    if "needs_layout_passes" in pltpu.CompilerParams.__dataclass_fields__:
        cp = dataclasses.replace(cp, needs_layout_passes=False)

    @functools.partial(
        pl.kernel,
        compiler_params=cp,
        out_type=jax.ShapeDtypeStruct((_T, _NPAD, _D), jnp.float32),
        mesh=mesh,
        scratch_types=[
            pltpu.VMEM((_NPAD,), jnp.float32),
            pltpu.VMEM((_CW,), jnp.int32),
            pltpu.VMEM((_CW,), jnp.int32),
            pltpu.VMEM((_LCAP,), jnp.int32),
            pltpu.VMEM((_LCAP,), jnp.float32),
            pltpu.VMEM((_C,), jnp.int32),
            pltpu.VMEM((_C,), jnp.int32),
            pltpu.VMEM((_C,), jnp.int32),
            pltpu.VMEM((_C,), jnp.int32),
            pltpu.VMEM((_C, _D), jnp.float32),
            pltpu.VMEM((_C, _D), jnp.float32),
            pltpu.VMEM((32, _D), jnp.float32),
            pltpu.VMEM_SHARED((_NPAD, _D), jnp.float32),
            pltpu.SemaphoreType.DMA,
            pltpu.SemaphoreType.DMA,
            pltpu.SemaphoreType.DMA,
            pltpu.SemaphoreType.DMA,
        ],
    )
    def kern(x_hbm, ed_hbm, nt_hbm, out_hbm,
             nt_v, ed0, ed1, sd_l, w_l, src0, src1, dst0, dst1, rows0, rows1,
             zero_v, acc, se0, se1, sg0, sg1):
        c = lax.axis_index("c")
        s = lax.axis_index("s")
        ed = (ed0, ed1)
        srcb = (src0, src1)
        dstb = (dst0, dst1)
        rows = (rows0, rows1)
        se = (se0, se1)
        sg = (sg0, sg1)
        zvec = jnp.zeros((16,), jnp.float32)
        iota = jnp.arange(16, dtype=jnp.int32)
        iota4 = iota * _REC
        trash_vec = jnp.full((16,), _TRASH, jnp.int32)

        @pl.loop(0, 32)
        def _(r):
            for j in range(_D // 16):
                zero_v[r, pl.ds(j * 16, 16)] = zvec

        chunk0 = s * _PER_SUB

        def issue_edata(q, b):
            return pltpu.async_copy(
                ed_hbm.at[pl.ds((chunk0 + q) * _CW, _CW)], ed[b], se[b])

        def wait_edata(b):
            pltpu.make_async_copy(
                ed_hbm.at[pl.ds(0, _CW)], ed[b], se[b]).wait()

        def unpack(k, b):
            for g in range(_C // 16):
                slk = pl.ds(g * 16, 16)
                v16 = plsc.load_gather(sd_l, [iota + (k * _C + g * 16)])
                srcb[b][slk] = v16 & 0xFFFF
                dstb[b][slk] = lax.shift_right_logical(v16, 16)

        def issue_gather(b):
            return pltpu.async_copy(x_hbm.at[srcb[b]], rows[b], sg[b])

        def wait_gather(b):
            pltpu.make_async_copy(x_hbm.at[srcb[b]], rows[b], sg[b]).wait()

        def scale(k, b):
            @pl.loop(0, _C, step=4)
            def _(e):
                for u in range(4):
                    ws = plsc.load_gather(
                        w_l, [jnp.full((16,), k * _C + e + u, jnp.int32)])
                    for j in range(_D // 16):
                        slj = pl.ds(j * 16, 16)
                        rows[b][e + u, slj] = rows[b][e + u, slj] * ws

        def scatter(b):
            pltpu.sync_copy(rows[b], acc.at[dstb[b]], add=True)

        @pl.loop(0, 2)
        def _(p):
            snap = c * 2 + p

            @pl.loop(0, _ROWS_PER_TILE // 32)
            def _(z):
                pltpu.sync_copy(
                    zero_v, acc.at[pl.ds(s * _ROWS_PER_TILE + z * 32, 32)])
            pltpu.sync_copy(nt_hbm.at[pl.ds(snap * _NPAD, _NPAD)], nt_v)
            plsc.subcore_barrier()

            @pl.loop(0, _NSEG)
            def _(seg):
                @pl.loop(0, _LCAP // 16)
                def _(i):
                    sl = pl.ds(i * 16, 16)
                    sd_l[sl] = trash_vec
                    w_l[sl] = zvec

                seg0 = seg * _SEGC
                issue_edata(seg0 + 0, 0)
                issue_edata(seg0 + 1, 1)

                def scan_chunk(k, b, cnt_vec):
                    wait_edata(b)

                    @pl.when(k + 2 < _SEGC)
                    def _():
                        issue_edata(seg0 + k + 2, b)

                    for g in range(_C // 16):
                        cidx = iota4 + (g * 16 * _REC)
                        src16 = plsc.load_gather(ed[b], [cidx])
                        dst16 = plsc.load_gather(ed[b], [cidx + 1])
                        t16 = plsc.bitcast(
                            plsc.load_gather(ed[b], [cidx + 2]), jnp.float32)
                        wt16 = plsc.bitcast(
                            plsc.load_gather(ed[b], [cidx + 3]), jnp.float32)
                        nt16 = plsc.load_gather(nt_v, [dst16])
                        w16 = wt16 * (t16 <= nt16).astype(jnp.float32)
                        mask = w16 != 0.0
                        packed = src16 | lax.shift_left(dst16, 16)
                        r16 = plsc.cumsum(mask.astype(jnp.int32))
                        pos = cnt_vec + r16 - 1
                        plsc.store_scatter(sd_l, [pos], packed, mask=mask)
                        plsc.store_scatter(w_l, [pos], w16, mask=mask)
                        cnt_vec = cnt_vec + plsc.all_reduce_population_count(
                            mask)
                    return cnt_vec

                def scan_pair(ka, cnt_vec):
                    cnt_vec = scan_chunk(2 * ka, 0, cnt_vec)
                    cnt_vec = scan_chunk(2 * ka + 1, 1, cnt_vec)
                    return cnt_vec

                cnt_vec = lax.fori_loop(
                    0, _SEGC // 2, scan_pair, jnp.zeros((16,), jnp.int32))
                cnt = jnp.max(cnt_vec)
                npairs = (cnt + 2 * _C - 1) // (2 * _C)

                @pl.when(npairs > 0)
                def _():
                    unpack(0, 0)
                    issue_gather(0)
                    unpack(1, 1)
                    issue_gather(1)

                def drain_pair(i, _carry):
                    for b in range(2):
                        k = 2 * i + b
                        wait_gather(b)
                        scale(k, b)
                        scatter(b)

                        @pl.when(k + 2 < 2 * npairs)
                        def _():
                            unpack(k + 2, b)
                            issue_gather(b)
                    return _carry

                lax.fori_loop(0, npairs, drain_pair, jnp.int32(0))

            plsc.subcore_barrier()
            sl_out = pl.ds(s * _ROWS_PER_TILE, _ROWS_PER_TILE)
            pltpu.sync_copy(acc.at[sl_out], out_hbm.at[snap, sl_out])
            plsc.subcore_barrier()

    return kern(x, edata, nt_flat)


def _tc_proj(agg2, W, b):
    M = agg2.shape[0]
    BM = 2000

    def body(a_ref, w_ref, b_ref, o_ref):
        o_ref[...] = (
            jnp.dot(a_ref[...], w_ref[...], preferred_element_type=jnp.float32)
            + b_ref[...]
        )

    return pl.pallas_call(
        body,
        grid=(M // BM,),
        in_specs=[
            pl.BlockSpec((BM, _D), lambda m: (m, 0)),
            pl.BlockSpec((_D, _D), lambda m: (0, 0)),
            pl.BlockSpec((1, _D), lambda m: (0, 0)),
        ],
        out_specs=pl.BlockSpec((BM, _D), lambda m: (m, 0)),
        out_shape=jax.ShapeDtypeStruct((M, _D), jnp.float32),
    )(agg2, W, b.reshape(1, _D))


@jax.jit
def kernel(x, edge_index, edge_time, node_time, edge_weight, W, b):
    nt_flat = jnp.pad(node_time, ((0, 0), (0, _NPAD - _N))).reshape(-1)
    edata = jnp.stack(
        [edge_index[0], edge_index[1],
         jax.lax.bitcast_convert_type(edge_time, jnp.int32),
         jax.lax.bitcast_convert_type(edge_weight, jnp.int32)],
        axis=1).reshape(-1)
    agg = _sc_agg(x, edata, nt_flat)
    agg = agg[:, :_N, :]
    out = _tc_proj(agg.reshape(_T * _N, _D), W, b)
    return out.reshape(_T, _N, _D)


# speedup vs baseline: 2.0800x; 2.0800x over previous
import dataclasses
import functools

import jax
import jax.numpy as jnp
from jax import lax
from jax.experimental import pallas as pl
from jax.experimental.pallas import tpu as pltpu
from jax.experimental.pallas import tpu_sc as plsc

_N = 10000
_NPAD = 10240
_E = 320000
_D = 128
_T = 4

_C = 80
_NSUB = 16
_ROWS_PER_TILE = _NPAD // _NSUB
_PER_SUB = _E // _NSUB // _C
_REC = 4
_CW = _C * _REC


def _sc_agg(x, edata, nt_flat):
    mesh = plsc.VectorSubcoreMesh(core_axis_name="c", subcore_axis_name="s")
    cp = pltpu.CompilerParams(use_tc_tiling_on_sc=False)
    if "needs_layout_passes" in pltpu.CompilerParams.__dataclass_fields__:
        cp = dataclasses.replace(cp, needs_layout_passes=False)

    @functools.partial(
        pl.kernel,
        compiler_params=cp,
        out_type=jax.ShapeDtypeStruct((_T, _NPAD, _D), jnp.float32),
        mesh=mesh,
        scratch_types=[
            pltpu.VMEM((_NPAD,), jnp.float32),
            pltpu.VMEM((_CW,), jnp.int32),
            pltpu.VMEM((_CW,), jnp.int32),
            pltpu.VMEM((_C,), jnp.int32),
            pltpu.VMEM((_C,), jnp.int32),
            pltpu.VMEM((_C,), jnp.int32),
            pltpu.VMEM((_C,), jnp.int32),
            pltpu.VMEM((_C,), jnp.float32),
            pltpu.VMEM((_C,), jnp.float32),
            pltpu.VMEM((_C, _D), jnp.float32),
            pltpu.VMEM((_C, _D), jnp.float32),
            pltpu.VMEM((64, _D), jnp.float32),
            pltpu.VMEM_SHARED((_NPAD, _D), jnp.float32),
            pltpu.SemaphoreType.DMA,
            pltpu.SemaphoreType.DMA,
            pltpu.SemaphoreType.DMA,
            pltpu.SemaphoreType.DMA,
            pltpu.SemaphoreType.DMA,
            pltpu.SemaphoreType.DMA,
        ],
    )
    def kern(x_hbm, ed_hbm, nt_hbm, out_hbm,
             nt_v, ed0, ed1, src0, src1, dst0, dst1, w0, w1, rows0, rows1,
             zero_v, acc, se0, se1, sg0, sg1, ss0, ss1):
        c = lax.axis_index("c")
        s = lax.axis_index("s")
        ed = (ed0, ed1)
        srcb = (src0, src1)
        dstb = (dst0, dst1)
        wb = (w0, w1)
        rows = (rows0, rows1)
        se = (se0, se1)
        sg = (sg0, sg1)
        ss = (ss0, ss1)
        zvec = jnp.zeros((16,), jnp.float32)
        iota4 = jnp.arange(16, dtype=jnp.int32) * _REC

        @pl.loop(0, 64)
        def _(r):
            for j in range(_D // 16):
                zero_v[r, pl.ds(j * 16, 16)] = zvec

        chunk0 = s * _PER_SUB

        def issue_edata(q, b):
            return pltpu.async_copy(
                ed_hbm.at[pl.ds((chunk0 + q) * _CW, _CW)], ed[b], se[b])

        def wait_edata(b):
            pltpu.make_async_copy(
                ed_hbm.at[pl.ds(0, _CW)], ed[b], se[b]).wait()

        def prep(b):
            for g in range(_C // 16):
                slk = pl.ds(g * 16, 16)
                src16 = ed[b][pl.ds(0 * _C + g * 16, 16)]
                dst16 = ed[b][pl.ds(1 * _C + g * 16, 16)]
                t16 = plsc.bitcast(ed[b][pl.ds(2 * _C + g * 16, 16)],
                                   jnp.float32)
                wt16 = plsc.bitcast(ed[b][pl.ds(3 * _C + g * 16, 16)],
                                    jnp.float32)
                nt16 = plsc.load_gather(nt_v, [dst16])
                srcb[b][slk] = src16
                dstb[b][slk] = dst16
                wb[b][slk] = wt16 * (t16 <= nt16).astype(jnp.float32)

        def issue_gather(b):
            return pltpu.async_copy(x_hbm.at[srcb[b]], rows[b], sg[b])

        def wait_gather(b):
            pltpu.make_async_copy(x_hbm.at[srcb[b]], rows[b], sg[b]).wait()

        def scale(b):
            @pl.loop(0, _C, step=4)
            def _(e):
                for u in range(4):
                    ws = plsc.load_gather(
                        wb[b], [jnp.full((16,), e + u, jnp.int32)])
                    for j in range(_D // 16):
                        slj = pl.ds(j * 16, 16)
                        rows[b][e + u, slj] = rows[b][e + u, slj] * ws

        def issue_scatter(b):
            pltpu.sync_copy(rows[b], acc.at[dstb[b]], add=True)

        def wait_scatter(b):
            pass

        @pl.loop(0, 2)
        def _(p):
            snap = c * 2 + p

            @pl.loop(0, _ROWS_PER_TILE // 64)
            def _(z):
                pltpu.sync_copy(
                    zero_v, acc.at[pl.ds(s * _ROWS_PER_TILE + z * 64, 64)])
            pltpu.sync_copy(nt_hbm.at[pl.ds(snap * _NPAD, _NPAD)], nt_v)
            plsc.subcore_barrier()

            issue_edata(0, 0)
            issue_edata(1, 1)
            wait_edata(0)
            prep(0)
            issue_gather(0)
            issue_edata(2, 0)
            wait_edata(1)
            prep(1)
            issue_gather(1)
            issue_edata(3, 1)
            wait_gather(0)
            scale(0)
            issue_scatter(0)

            @pl.loop(0, (_PER_SUB - 2) // 2)
            def _(it):
                for b in range(2):
                    q = 2 * it + 2 + b
                    nb = 1 - b
                    wait_edata(b)
                    wait_scatter(b)
                    prep(b)
                    issue_gather(b)

                    @pl.when(q + 2 < _PER_SUB)
                    def _():
                        issue_edata(q + 2, b)

                    wait_gather(nb)
                    scale(nb)
                    issue_scatter(nb)

            wait_gather(1)
            scale(1)
            issue_scatter(1)
            wait_scatter(0)
            wait_scatter(1)

            plsc.subcore_barrier()
            sl_out = pl.ds(s * _ROWS_PER_TILE, _ROWS_PER_TILE)
            pltpu.sync_copy(acc.at[sl_out], out_hbm.at[snap, sl_out])
            plsc.subcore_barrier()

    return kern(x, edata, nt_flat)


def _tc_proj(agg2, W, b):
    M = agg2.shape[0]
    BM = 2000

    def body(a_ref, w_ref, b_ref, o_ref):
        o_ref[...] = (
            jnp.dot(a_ref[...], w_ref[...], preferred_element_type=jnp.float32)
            + b_ref[...]
        )

    return pl.pallas_call(
        body,
        grid=(M // BM,),
        in_specs=[
            pl.BlockSpec((BM, _D), lambda m: (m, 0)),
            pl.BlockSpec((_D, _D), lambda m: (0, 0)),
            pl.BlockSpec((1, _D), lambda m: (0, 0)),
        ],
        out_specs=pl.BlockSpec((BM, _D), lambda m: (m, 0)),
        out_shape=jax.ShapeDtypeStruct((M, _D), jnp.float32),
    )(agg2, W, b.reshape(1, _D))


@jax.jit
def kernel(x, edge_index, edge_time, node_time, edge_weight, W, b):
    nt_flat = jnp.pad(node_time, ((0, 0), (0, _NPAD - _N))).reshape(-1)
    edata = jnp.stack(
        [edge_index[0].reshape(_E // _C, _C),
         edge_index[1].reshape(_E // _C, _C),
         jax.lax.bitcast_convert_type(edge_time, jnp.int32).reshape(
             _E // _C, _C),
         jax.lax.bitcast_convert_type(edge_weight, jnp.int32).reshape(
             _E // _C, _C)],
        axis=1).reshape(-1)
    agg = _sc_agg(x, edata, nt_flat)
    agg = agg[:, :_N, :]
    out = _tc_proj(agg.reshape(_T * _N, _D), W, b)
    return out.reshape(_T, _N, _D)


# speedup vs baseline: 2.4114x; 1.1593x over previous
import dataclasses
import functools

import jax
import jax.numpy as jnp
from jax import lax
from jax.experimental import pallas as pl
from jax.experimental.pallas import tpu as pltpu
from jax.experimental.pallas import tpu_sc as plsc

_N = 10000
_NPAD = 10240
_E = 320000
_D = 128
_T = 4

_C = 80
_NSUB = 16
_ROWS_PER_TILE = _NPAD // _NSUB
_PER_SUB = _E // _NSUB // _C
_REC = 4
_CW = _C * _REC


def _sc_agg(x, edata, nt_flat):
    mesh = plsc.VectorSubcoreMesh(core_axis_name="c", subcore_axis_name="s")
    cp = pltpu.CompilerParams(use_tc_tiling_on_sc=False)
    if "needs_layout_passes" in pltpu.CompilerParams.__dataclass_fields__:
        cp = dataclasses.replace(cp, needs_layout_passes=False)

    @functools.partial(
        pl.kernel,
        compiler_params=cp,
        out_type=jax.ShapeDtypeStruct((_T, _NPAD, _D), jnp.float32),
        mesh=mesh,
        scratch_types=[
            pltpu.VMEM((_NPAD,), jnp.float32),
            pltpu.VMEM((_CW,), jnp.int32),
            pltpu.VMEM((_CW,), jnp.int32),
            pltpu.VMEM((_C,), jnp.int32),
            pltpu.VMEM((_C,), jnp.int32),
            pltpu.VMEM((_C,), jnp.int32),
            pltpu.VMEM((_C,), jnp.int32),
            pltpu.VMEM((_C,), jnp.float32),
            pltpu.VMEM((_C,), jnp.float32),
            pltpu.VMEM((_C, _D), jnp.float32),
            pltpu.VMEM((_C, _D), jnp.float32),
            pltpu.VMEM((64, _D), jnp.float32),
            pltpu.VMEM_SHARED((_NPAD, _D), jnp.float32),
            pltpu.SemaphoreType.DMA,
            pltpu.SemaphoreType.DMA,
            pltpu.SemaphoreType.DMA,
            pltpu.SemaphoreType.DMA,
            pltpu.SemaphoreType.DMA,
            pltpu.SemaphoreType.DMA,
        ],
    )
    def kern(x_hbm, ed_hbm, nt_hbm, out_hbm,
             nt_v, ed0, ed1, src0, src1, dst0, dst1, w0, w1, rows0, rows1,
             zero_v, acc, se0, se1, sg0, sg1, ss0, ss1):
        c = lax.axis_index("c")
        s = lax.axis_index("s")
        ed = (ed0, ed1)
        srcb = (src0, src1)
        dstb = (dst0, dst1)
        wb = (w0, w1)
        rows = (rows0, rows1)
        se = (se0, se1)
        sg = (sg0, sg1)
        ss = (ss0, ss1)
        zvec = jnp.zeros((16,), jnp.float32)
        iota = jnp.arange(16, dtype=jnp.int32)

        @pl.loop(0, 64)
        def _(r):
            for j in range(_D // 16):
                zero_v[r, pl.ds(j * 16, 16)] = zvec

        chunk0 = s * _PER_SUB

        def issue_edata(q, b):
            return pltpu.async_copy(
                ed_hbm.at[pl.ds((chunk0 + q) * _CW, _CW)], ed[b], se[b])

        def wait_edata(b):
            pltpu.make_async_copy(
                ed_hbm.at[pl.ds(0, _CW)], ed[b], se[b]).wait()

        def prep(b):
            for g in range(_C // 16):
                slk = pl.ds(g * 16, 16)
                src16 = ed[b][pl.ds(0 * _C + g * 16, 16)]
                dst16 = ed[b][pl.ds(1 * _C + g * 16, 16)]
                t16 = plsc.bitcast(ed[b][pl.ds(2 * _C + g * 16, 16)],
                                   jnp.float32)
                wt16 = plsc.bitcast(ed[b][pl.ds(3 * _C + g * 16, 16)],
                                    jnp.float32)
                nt16 = plsc.load_gather(nt_v, [dst16])
                srcb[b][slk] = src16
                dstb[b][slk] = dst16
                wb[b][slk] = wt16 * (t16 <= nt16).astype(jnp.float32)

        def issue_gather(b):
            return pltpu.async_copy(x_hbm.at[srcb[b]], rows[b], sg[b])

        def wait_gather(b):
            pltpu.make_async_copy(x_hbm.at[srcb[b]], rows[b], sg[b]).wait()

        def scale(b):
            @pl.loop(0, _C, step=16)
            def _(e):
                wvec = plsc.load_gather(wb[b], [iota + e])
                for u in range(16):
                    ws = wvec.at[jnp.full((16,), u, jnp.int32)].get(
                        mode="promise_in_bounds")
                    for j in range(_D // 16):
                        slj = pl.ds(j * 16, 16)
                        rows[b][e + u, slj] = rows[b][e + u, slj] * ws

        def issue_scatter(b):
            pltpu.sync_copy(rows[b], acc.at[dstb[b]], add=True)

        def wait_scatter(b):
            pass

        @pl.loop(0, 2)
        def _(p):
            snap = c * 2 + p

            @pl.loop(0, _ROWS_PER_TILE // 64)
            def _(z):
                pltpu.sync_copy(
                    zero_v, acc.at[pl.ds(s * _ROWS_PER_TILE + z * 64, 64)])
            pltpu.sync_copy(nt_hbm.at[pl.ds(snap * _NPAD, _NPAD)], nt_v)
            plsc.subcore_barrier()

            issue_edata(0, 0)
            issue_edata(1, 1)
            wait_edata(0)
            prep(0)
            issue_gather(0)
            issue_edata(2, 0)
            wait_edata(1)
            prep(1)
            issue_gather(1)
            issue_edata(3, 1)
            wait_gather(0)
            scale(0)
            issue_scatter(0)

            @pl.loop(0, (_PER_SUB - 2) // 2)
            def _(it):
                for b in range(2):
                    q = 2 * it + 2 + b
                    nb = 1 - b
                    wait_edata(b)
                    wait_scatter(b)
                    prep(b)
                    issue_gather(b)

                    @pl.when(q + 2 < _PER_SUB)
                    def _():
                        issue_edata(q + 2, b)

                    wait_gather(nb)
                    scale(nb)
                    issue_scatter(nb)

            wait_gather(1)
            scale(1)
            issue_scatter(1)
            wait_scatter(0)
            wait_scatter(1)

            plsc.subcore_barrier()
            sl_out = pl.ds(s * _ROWS_PER_TILE, _ROWS_PER_TILE)
            pltpu.sync_copy(acc.at[sl_out], out_hbm.at[snap, sl_out])
            plsc.subcore_barrier()

    return kern(x, edata, nt_flat)


def _tc_proj(agg2, W, b):
    M = agg2.shape[0]
    BM = 2000

    def body(a_ref, w_ref, b_ref, o_ref):
        o_ref[...] = (
            jnp.dot(a_ref[...], w_ref[...], preferred_element_type=jnp.float32)
            + b_ref[...]
        )

    return pl.pallas_call(
        body,
        grid=(M // BM,),
        in_specs=[
            pl.BlockSpec((BM, _D), lambda m: (m, 0)),
            pl.BlockSpec((_D, _D), lambda m: (0, 0)),
            pl.BlockSpec((1, _D), lambda m: (0, 0)),
        ],
        out_specs=pl.BlockSpec((BM, _D), lambda m: (m, 0)),
        out_shape=jax.ShapeDtypeStruct((M, _D), jnp.float32),
    )(agg2, W, b.reshape(1, _D))


@jax.jit
def kernel(x, edge_index, edge_time, node_time, edge_weight, W, b):
    nt_flat = jnp.pad(node_time, ((0, 0), (0, _NPAD - _N))).reshape(-1)
    edata = jnp.stack(
        [edge_index[0].reshape(_E // _C, _C),
         edge_index[1].reshape(_E // _C, _C),
         jax.lax.bitcast_convert_type(edge_time, jnp.int32).reshape(
             _E // _C, _C),
         jax.lax.bitcast_convert_type(edge_weight, jnp.int32).reshape(
             _E // _C, _C)],
        axis=1).reshape(-1)
    agg = _sc_agg(x, edata, nt_flat)
    agg = agg[:, :_N, :]
    out = _tc_proj(agg.reshape(_T * _N, _D), W, b)
    return out.reshape(_T, _N, _D)
